# baseline (speedup 1.0000x reference)
import functools

import jax
import jax.numpy as jnp
from jax import lax
from jax.experimental import pallas as pl
from jax.experimental.pallas import tpu as pltpu
from jax.experimental.pallas import tpu_sc as plsc

N, E, B, L, K, M, R, D, F = 10000, 320000, 1024, 2, 64, 100000, 1000, 128, 128

NC, NS = 2, 16
NW = NC * NS
EPT = E // NW
CHE = 128
NCHE = EPT // CHE
TAIL = EPT - NCHE * CHE
MP = 128

_mesh = plsc.VectorSubcoreMesh(
    core_axis_name="c", subcore_axis_name="s", num_cores=NC, num_subcores=NS)


def _tile_ids():
  c = lax.axis_index("c")
  s = lax.axis_index("s")
  return c, s, c * NS + s


def _fill_rows(ref, nrows, ncols, value):
  v = jnp.full((16,), value, dtype=jnp.float32)

  def body(i, _):
    for col in range(ncols // 16):
      ref[i, pl.ds(col * 16, 16)] = v
    return 0

  lax.fori_loop(0, nrows, body, 0)


def _per_sc_row_blocks(s, nrows, blk, fn):
  nb = nrows // blk

  def body(i, _):
    j = i * NS + s

    @pl.when(j < nb)
    def _():
      fn(j * blk)

    return 0

  lax.fori_loop(0, (nb + NS - 1) // NS, body, 0)


@functools.partial(
    pl.kernel,
    out_type=jax.ShapeDtypeStruct((NW, 2, N), jnp.float32),
    mesh=_mesh,
    scratch_types=[
        pltpu.VMEM((EPT,), jnp.int32),
        pltpu.VMEM((EPT,), jnp.int32),
        pltpu.VMEM((N,), jnp.float32),
        pltpu.VMEM((N,), jnp.float32),
    ],
    compiler_params=pltpu.CompilerParams(needs_layout_passes=False),
)
def _sc_degrees(src_hbm, dst_hbm, out, sidx, didx, dsrc, ddst):
  _, _, tile = _tile_ids()
  zero16 = jnp.zeros((16,), jnp.float32)
  ones16 = jnp.ones((16,), jnp.float32)

  def zbody(i, _):
    dsrc[pl.ds(i * 16, 16)] = zero16
    ddst[pl.ds(i * 16, 16)] = zero16
    return 0

  lax.fori_loop(0, N // 16, zbody, 0)
  base = tile * EPT
  pltpu.sync_copy(src_hbm.at[pl.ds(base, EPT)], sidx)
  pltpu.sync_copy(dst_hbm.at[pl.ds(base, EPT)], didx)

  def body(k, _):
    plsc.addupdate_scatter(dsrc, [sidx[pl.ds(k * 16, 16)]], ones16)
    plsc.addupdate_scatter(ddst, [didx[pl.ds(k * 16, 16)]], ones16)
    return 0

  lax.fori_loop(0, EPT // 16, body, 0)
  pltpu.sync_copy(dsrc, out.at[tile].at[0])
  pltpu.sync_copy(ddst, out.at[tile].at[1])


CHP = 80
NCHP = EPT // CHP


@functools.partial(
    pl.kernel,
    out_type=jax.ShapeDtypeStruct((NC, N, D), jnp.float32),
    mesh=_mesh,
    scratch_types=[
        pltpu.VMEM((EPT,), jnp.int32),
        pltpu.VMEM((EPT,), jnp.int32),
        pltpu.VMEM((CHP, D), jnp.float32),
        pltpu.VMEM((CHP, D), jnp.float32),
        pltpu.VMEM((CHP, D), jnp.float32),
        pltpu.VMEM_SHARED((N, D), jnp.float32),
        pltpu.SemaphoreType.DMA,
        pltpu.SemaphoreType.DMA,
        pltpu.SemaphoreType.DMA,
        pltpu.SemaphoreType.DMA,
        pltpu.SemaphoreType.DMA,
        pltpu.SemaphoreType.DMA,
    ],
)
def _sc_edge_pass(m_hbm, src_hbm, dst_hbm, out, srcall, dstall, rows0, rows1,
                  rows2, acc_sp, sg0, sg1, sg2, ss0, ss1, ss2):
  c, s, tile = _tile_ids()
  rowsb = (rows0, rows1, rows2)
  sg = (sg0, sg1, sg2)
  ss = (ss0, ss1, ss2)
  _fill_rows(rows0, 16, D, 0.0)
  _per_sc_row_blocks(
      s, N, 16,
      lambda off: pltpu.sync_copy(rows0.at[pl.ds(0, 16)],
                                  acc_sp.at[pl.ds(off, 16)]))
  plsc.subcore_barrier()
  base = tile * EPT
  pltpu.sync_copy(src_hbm.at[pl.ds(base, EPT)], srcall)
  pltpu.sync_copy(dst_hbm.at[pl.ds(base, EPT)], dstall)
  for b in range(3):
    pltpu.async_copy(m_hbm.at[srcall.at[pl.ds(b * CHP, CHP)]], rowsb[b],
                     sg[b])

  def chunk(cn, b):
    pltpu.make_async_copy(m_hbm.at[pl.ds(0, CHP)], rowsb[b], sg[b]).wait()
    pltpu.async_copy(rowsb[b], acc_sp.at[dstall.at[pl.ds(cn * CHP, CHP)]],
                     ss[b], add=True)

    @pl.when(cn + 3 < NCHP)
    def _():
      pltpu.make_async_copy(rowsb[b], acc_sp.at[pl.ds(0, CHP)], ss[b]).wait()
      pltpu.async_copy(m_hbm.at[srcall.at[pl.ds((cn + 3) * CHP, CHP)]],
                       rowsb[b], sg[b])

  def group(g, _):
    for b in range(3):
      chunk(g * 3 + b, b)
    return 0

  lax.fori_loop(0, NCHP // 3, group, 0)
  for b in range(NCHP - (NCHP // 3) * 3):
    chunk((NCHP // 3) * 3 + b, b)
  for b in range(3):
    pltpu.make_async_copy(rowsb[b], acc_sp.at[pl.ds(0, CHP)], ss[b]).wait()
  plsc.subcore_barrier()
  _per_sc_row_blocks(
      s, N, 80,
      lambda off: pltpu.sync_copy(acc_sp.at[pl.ds(off, 80)],
                                  out.at[c].at[pl.ds(off, 80)]))


@functools.partial(
    pl.kernel,
    out_type=jax.ShapeDtypeStruct((NC, B, D), jnp.float32),
    mesh=_mesh,
    scratch_types=[
        pltpu.VMEM((80,), jnp.int32),
        pltpu.VMEM((80, D), jnp.float32),
        pltpu.VMEM((B // NS, D), jnp.float32),
        pltpu.VMEM_SHARED((B, D), jnp.float32),
    ],
)
def _sc_atom_agg(h_hbm, gids_hbm, out, sidx, rows, zbuf, acc_sp):
  c, s, tile = _tile_ids()
  bpt = B // NS
  _fill_rows(zbuf, bpt, D, 0.0)
  pltpu.sync_copy(zbuf, acc_sp.at[pl.ds(s * bpt, bpt)])
  plsc.subcore_barrier()

  nblocks = N // 80

  def body(i, _):
    j = i * NW + tile

    @pl.when(j < nblocks)
    def _():
      pltpu.sync_copy(h_hbm.at[pl.ds(j * 80, 80)], rows)
      pltpu.sync_copy(gids_hbm.at[pl.ds(j * 80, 80)], sidx)
      pltpu.sync_copy(rows, acc_sp.at[sidx], add=True)

    return 0

  lax.fori_loop(0, (nblocks + NW - 1) // NW, body, 0)
  plsc.subcore_barrier()
  pltpu.sync_copy(acc_sp.at[pl.ds(s * bpt, bpt)],
                  out.at[c].at[pl.ds(s * bpt, bpt)])


_IPT = (L * B) // NW


@functools.partial(
    pl.kernel,
    out_type=(
        jax.ShapeDtypeStruct((L * B, 128), jnp.int32),
        jax.ShapeDtypeStruct((L * B, MP), jnp.float32),
    ),
    mesh=_mesh,
    scratch_types=[
        pltpu.VMEM((_IPT,), jnp.int32),
        pltpu.VMEM((_IPT,), jnp.int32),
        pltpu.VMEM((_IPT, 128), jnp.int32),
        pltpu.VMEM((_IPT, MP), jnp.float32),
        pltpu.SemaphoreType.DMA,
    ],
)
def _sc_kg_stage1(idflat, idhalf, nm2_hbm, mfea_hbm, out_nei, out_item, idbuf,
                  idhbuf, nmrows, mrows, sem):
  _, _, tile = _tile_ids()
  i0 = tile * _IPT
  pltpu.sync_copy(idflat.at[pl.ds(i0, _IPT)], idbuf)
  pltpu.sync_copy(idhalf.at[pl.ds(i0, _IPT)], idhbuf)
  pltpu.async_copy(nm2_hbm.at[idhbuf], nmrows, sem).wait()
  pltpu.sync_copy(nmrows, out_nei.at[pl.ds(i0, _IPT)])
  pltpu.async_copy(mfea_hbm.at[idbuf], mrows, sem).wait()
  pltpu.sync_copy(mrows, out_item.at[pl.ds(i0, _IPT)])


_NFLAT = L * B * K
_NPT = _NFLAT // NW
_CH2 = 128
_NCH2 = _NPT // _CH2
_ACC2 = L * B


@functools.partial(
    pl.kernel,
    out_type=jax.ShapeDtypeStruct((NC, L * B, MP), jnp.float32),
    mesh=_mesh,
    scratch_types=[
        pltpu.VMEM((_NPT,), jnp.int32),
        pltpu.VMEM((_NPT,), jnp.int32),
        pltpu.VMEM((_CH2, MP), jnp.float32),
        pltpu.VMEM((_CH2, MP), jnp.float32),
        pltpu.VMEM((_CH2, MP), jnp.float32),
        pltpu.VMEM_SHARED((_ACC2, MP), jnp.float32),
        pltpu.SemaphoreType.DMA,
        pltpu.SemaphoreType.DMA,
        pltpu.SemaphoreType.DMA,
        pltpu.SemaphoreType.DMA,
        pltpu.SemaphoreType.DMA,
        pltpu.SemaphoreType.DMA,
    ],
)
def _sc_kg_stage2(neiflat, pat_hbm, mfea_hbm, out, gall, pall, rows0, rows1,
                  rows2, acc_sp, sg0, sg1, sg2, ss0, ss1, ss2):
  c, s, tile = _tile_ids()
  rowsb = (rows0, rows1, rows2)
  sg = (sg0, sg1, sg2)
  ss = (ss0, ss1, ss2)
  _fill_rows(rows0, 8, MP, 0.0)
  _per_sc_row_blocks(
      s, _ACC2, 8,
      lambda off: pltpu.sync_copy(rows0.at[pl.ds(0, 8)],
                                  acc_sp.at[pl.ds(off, 8)]))
  plsc.subcore_barrier()
  base = tile * _NPT
  pltpu.sync_copy(neiflat.at[pl.ds(base, _NPT)], gall)
  pltpu.sync_copy(pat_hbm.at[pl.ds(base, _NPT)], pall)
  for b in range(3):
    pltpu.async_copy(mfea_hbm.at[gall.at[pl.ds(b * _CH2, _CH2)]], rowsb[b],
                     sg[b])

  def chunk(cn, b):
    pltpu.make_async_copy(mfea_hbm.at[pl.ds(0, _CH2)], rowsb[b], sg[b]).wait()
    pltpu.async_copy(rowsb[b], acc_sp.at[pall.at[pl.ds(cn * _CH2, _CH2)]],
                     ss[b], add=True)

    @pl.when(cn + 3 < _NCH2)
    def _():
      pltpu.make_async_copy(rowsb[b], acc_sp.at[pl.ds(0, _CH2)], ss[b]).wait()
      pltpu.async_copy(mfea_hbm.at[gall.at[pl.ds((cn + 3) * _CH2, _CH2)]],
                       rowsb[b], sg[b])

  def group(g, _):
    for b in range(3):
      chunk(g * 3 + b, b)
    return 0

  lax.fori_loop(0, _NCH2 // 3, group, 0)
  for b in range(_NCH2 - (_NCH2 // 3) * 3):
    chunk((_NCH2 // 3) * 3 + b, b)
  for b in range(3):
    pltpu.make_async_copy(rowsb[b], acc_sp.at[pl.ds(0, _CH2)], ss[b]).wait()
  plsc.subcore_barrier()
  rpt = (L * B) // NS
  pltpu.sync_copy(acc_sp.at[pl.ds(s * rpt, rpt)],
                  out.at[c].at[pl.ds(s * rpt, rpt)])


_MB = 2000


def _tc_embed_body(mf_ref, fe_ref, out_ref):
  out_ref[...] = jnp.dot(mf_ref[...], fe_ref[...],
                         preferred_element_type=jnp.float32)


def _tc_embed(all_mfea, fea_emb):
  return pl.pallas_call(
      _tc_embed_body,
      grid=(M // _MB,),
      in_specs=[
          pl.BlockSpec((_MB, 40), lambda i: (i, 0)),
          pl.BlockSpec((40, D), lambda i: (0, 0)),
      ],
      out_specs=pl.BlockSpec((_MB, D), lambda i: (i, 0)),
      out_shape=jax.ShapeDtypeStruct((M, D), jnp.float32),
  )(all_mfea, fea_emb)


def _tc_prep_body(nf_ref, degp_ref, w0_ref, m1_ref, ns_ref, nd_ref):
  dsrc = jnp.sum(degp_ref[:, 0, :], axis=0)
  ddst = jnp.sum(degp_ref[:, 1, :], axis=0)
  ns = jax.lax.rsqrt(jnp.maximum(dsrc, 1.0))[:, None]
  nd = jax.lax.rsqrt(jnp.maximum(ddst, 1.0))[:, None]
  iota = lax.broadcasted_iota(jnp.int32, (N, F), 1)
  h0 = jnp.zeros((N, F), jnp.float32)
  for j in range(4):
    h0 = h0 + (nf_ref[:, j:j + 1] == iota).astype(jnp.float32)
  m1_ref[...] = jnp.dot(h0 * ns, w0_ref[...],
                        preferred_element_type=jnp.float32)
  ns_ref[...] = ns
  nd_ref[...] = nd


def _tc_prep(node_feature, degp, w0):
  return pl.pallas_call(
      _tc_prep_body,
      out_shape=[
          jax.ShapeDtypeStruct((N, D), jnp.float32),
          jax.ShapeDtypeStruct((N, 1), jnp.float32),
          jax.ShapeDtypeStruct((N, 1), jnp.float32),
      ],
  )(node_feature, degp, w0)


def _tc_mid_body(aggp_ref, nd_ref, ns_ref, b0_ref, w1_ref, m2_ref):
  agg = aggp_ref[0] + aggp_ref[1]
  h1 = jnp.maximum(agg * nd_ref[...] + b0_ref[...], 0.0)
  m2_ref[...] = jnp.dot(h1 * ns_ref[...], w1_ref[...],
                        preferred_element_type=jnp.float32)


def _tc_mid(aggp, nd, ns, b0, w1):
  return pl.pallas_call(
      _tc_mid_body,
      out_shape=jax.ShapeDtypeStruct((N, D), jnp.float32),
  )(aggp, nd, ns, b0, w1)


def _tc_h2_body(aggp_ref, nd_ref, b1_ref, h2_ref, fac_ref):
  agg = aggp_ref[0] + aggp_ref[1]
  h2 = agg * nd_ref[...] + b1_ref[...]
  h2_ref[...] = h2
  nsum = jnp.sum(jnp.sqrt(jnp.sum(h2 * h2, axis=1)))
  fac_ref[...] = jnp.full((1, 1), jnp.sqrt(jnp.float32(D)) * N / nsum)


def _tc_h2(aggp, nd, b1):
  return pl.pallas_call(
      _tc_h2_body,
      out_shape=[
          jax.ShapeDtypeStruct((N, D), jnp.float32),
          jax.ShapeDtypeStruct((1, 1), jnp.float32),
      ],
  )(aggp, nd, b1)


def _tc_nm2_body(nm3_ref, out_ref):
  out_ref[...] = jnp.concatenate([nm3_ref[:, 0, :], nm3_ref[:, 1, :]], axis=1)


def _tc_nm2(neighbor_matrix):
  return pl.pallas_call(
      _tc_nm2_body,
      grid=(M // 2000,),
      in_specs=[pl.BlockSpec((1000, 2, K), lambda i: (i, 0, 0))],
      out_specs=pl.BlockSpec((1000, 2 * K), lambda i: (i, 0)),
      out_shape=jax.ShapeDtypeStruct(((M * K) // 128, 128), jnp.int32),
  )(neighbor_matrix.reshape(M // 2, 2, K))


def _tc_compact_body(nei_ref, par_ref, out_ref):
  sel = par_ref[...] == 1
  out_ref[...] = jnp.where(sel, nei_ref[:, K:], nei_ref[:, :K])


def _tc_compact(nei_rows, par):
  return pl.pallas_call(
      _tc_compact_body,
      out_shape=jax.ShapeDtypeStruct((L * B, K), jnp.int32),
  )(nei_rows, par)


_BB = 256


def _tc_final_body(molap_ref, factor_ref, item_ref, neip_ref, cw_ref,
                   cb_ref, emb_ref, a_ref, x_ref):
  mol_r = (item_ref[0] + item_ref[1] + neip_ref[0, 0] + neip_ref[0, 1] +
           neip_ref[1, 0] + neip_ref[1, 1]) * 0.5
  mol_a = (molap_ref[0] + molap_ref[1]) * factor_ref[0, 0]
  emb_ref[...] = mol_a + mol_r
  a_ref[...] = mol_a
  x_ref[...] = jnp.dot(mol_r, cw_ref[...],
                       preferred_element_type=jnp.float32) + cb_ref[...]


def _tc_final(molap, factor, item2, neip, cw, cb):
  return pl.pallas_call(
      _tc_final_body,
      grid=(B // _BB,),
      in_specs=[
          pl.BlockSpec((NC, _BB, D), lambda i: (0, i, 0)),
          pl.BlockSpec((1, 1), lambda i: (0, 0)),
          pl.BlockSpec((L, _BB, MP), lambda i: (0, i, 0)),
          pl.BlockSpec((NC, L, _BB, MP), lambda i: (0, 0, i, 0)),
          pl.BlockSpec((D, D), lambda i: (0, 0)),
          pl.BlockSpec((1, D), lambda i: (0, 0)),
      ],
      out_specs=[
          pl.BlockSpec((_BB, D), lambda i: (i, 0)),
          pl.BlockSpec((_BB, D), lambda i: (i, 0)),
          pl.BlockSpec((_BB, D), lambda i: (i, 0)),
      ],
      out_shape=[
          jax.ShapeDtypeStruct((B, D), jnp.float32),
          jax.ShapeDtypeStruct((B, D), jnp.float32),
          jax.ShapeDtypeStruct((B, D), jnp.float32),
      ],
  )(molap, factor, item2, neip, cw, cb)


@jax.jit
def kernel(node_feature, edge_index, graph_ids, id_list, id_list_target,
           neighbor_matrix, relation_matrix, all_mfea, all_rfea,
           W0, b0, W1, b1, fea_emb, rel_emb, crosscl_w, crosscl_b):
  del id_list_target, relation_matrix, all_rfea, rel_emb

  src = edge_index[0].astype(jnp.int32)
  dst = edge_index[1].astype(jnp.int32)
  gids = graph_ids.astype(jnp.int32)
  idflat = id_list.astype(jnp.int32).reshape(L * B)
  idhalf = idflat // 2
  nm2 = _tc_nm2(neighbor_matrix.astype(jnp.int32))
  mfea_emb = _tc_embed(all_mfea, fea_emb)
  pat = jnp.repeat(jnp.arange(L * B, dtype=jnp.int32), K)

  degp = _sc_degrees(src, dst)
  m1, ns, nd = _tc_prep(node_feature.astype(jnp.int32), degp, W0)
  agg1p = _sc_edge_pass(m1, src, dst)
  m2 = _tc_mid(agg1p, nd, ns, b0.reshape(1, D), W1)
  agg2p = _sc_edge_pass(m2, src, dst)
  h2, factor = _tc_h2(agg2p, nd, b1.reshape(1, D))
  molap = _sc_atom_agg(h2, gids)

  nei_rows, item_rows = _sc_kg_stage1(idflat, idhalf, nm2, mfea_emb)
  nei_compact = _tc_compact(nei_rows, (idflat % 2).reshape(L * B, 1))
  neip = _sc_kg_stage2(nei_compact.reshape(_NFLAT), pat, mfea_emb)

  mol_emb, mol_a, mol_x = _tc_final(
      molap, factor, item_rows.reshape(L, B, MP),
      neip.reshape(NC, L, B, MP), crosscl_w, crosscl_b.reshape(1, D))
  return (mol_emb, mol_a, mol_x)


# speedup vs baseline: 1.0295x; 1.0295x over previous
import functools

import jax
import jax.numpy as jnp
from jax import lax
from jax.experimental import pallas as pl
from jax.experimental.pallas import tpu as pltpu
from jax.experimental.pallas import tpu_sc as plsc

N, E, B, L, K, M, R, D, F = 10000, 320000, 1024, 2, 64, 100000, 1000, 128, 128

NC, NS = 2, 16
NW = NC * NS
EPT = E // NW
CHE = 128
NCHE = EPT // CHE
TAIL = EPT - NCHE * CHE
MP = 128

_mesh = plsc.VectorSubcoreMesh(
    core_axis_name="c", subcore_axis_name="s", num_cores=NC, num_subcores=NS)


def _tile_ids():
  c = lax.axis_index("c")
  s = lax.axis_index("s")
  return c, s, c * NS + s


def _fill_rows(ref, nrows, ncols, value):
  v = jnp.full((16,), value, dtype=jnp.float32)

  def body(i, _):
    for col in range(ncols // 16):
      ref[i, pl.ds(col * 16, 16)] = v
    return 0

  lax.fori_loop(0, nrows, body, 0)


def _per_sc_row_blocks(s, nrows, blk, fn):
  nb = nrows // blk

  def body(i, _):
    j = i * NS + s

    @pl.when(j < nb)
    def _():
      fn(j * blk)

    return 0

  lax.fori_loop(0, (nb + NS - 1) // NS, body, 0)


@functools.partial(
    pl.kernel,
    out_type=jax.ShapeDtypeStruct((NW, 2, N), jnp.float32),
    mesh=_mesh,
    scratch_types=[
        pltpu.VMEM((EPT,), jnp.int32),
        pltpu.VMEM((EPT,), jnp.int32),
        pltpu.VMEM((N,), jnp.float32),
        pltpu.VMEM((N,), jnp.float32),
    ],
    compiler_params=pltpu.CompilerParams(needs_layout_passes=False),
)
def _sc_degrees(src_hbm, dst_hbm, out, sidx, didx, dsrc, ddst):
  _, _, tile = _tile_ids()
  zero16 = jnp.zeros((16,), jnp.float32)
  ones16 = jnp.ones((16,), jnp.float32)

  def zbody(i, _):
    dsrc[pl.ds(i * 16, 16)] = zero16
    ddst[pl.ds(i * 16, 16)] = zero16
    return 0

  lax.fori_loop(0, N // 16, zbody, 0)
  base = tile * EPT
  pltpu.sync_copy(src_hbm.at[pl.ds(base, EPT)], sidx)
  pltpu.sync_copy(dst_hbm.at[pl.ds(base, EPT)], didx)

  def body(k, _):
    plsc.addupdate_scatter(dsrc, [sidx[pl.ds(k * 16, 16)]], ones16)
    plsc.addupdate_scatter(ddst, [didx[pl.ds(k * 16, 16)]], ones16)
    return 0

  lax.fori_loop(0, EPT // 16, body, 0)
  pltpu.sync_copy(dsrc, out.at[tile].at[0])
  pltpu.sync_copy(ddst, out.at[tile].at[1])


CHP = 80
NCHP = EPT // CHP


@functools.partial(
    pl.kernel,
    out_type=jax.ShapeDtypeStruct((NC, N, D), jnp.float32),
    mesh=_mesh,
    scratch_types=[
        pltpu.VMEM((EPT,), jnp.int32),
        pltpu.VMEM((EPT,), jnp.int32),
        pltpu.VMEM((CHP, D), jnp.float32),
        pltpu.VMEM((CHP, D), jnp.float32),
        pltpu.VMEM((CHP, D), jnp.float32),
        pltpu.VMEM_SHARED((N, D), jnp.float32),
        pltpu.SemaphoreType.DMA,
        pltpu.SemaphoreType.DMA,
        pltpu.SemaphoreType.DMA,
        pltpu.SemaphoreType.DMA,
        pltpu.SemaphoreType.DMA,
        pltpu.SemaphoreType.DMA,
    ],
)
def _sc_edge_pass(m_hbm, src_hbm, dst_hbm, out, srcall, dstall, rows0, rows1,
                  rows2, acc_sp, sg0, sg1, sg2, ss0, ss1, ss2):
  c, s, tile = _tile_ids()
  rowsb = (rows0, rows1, rows2)
  sg = (sg0, sg1, sg2)
  ss = (ss0, ss1, ss2)
  _fill_rows(rows0, 16, D, 0.0)
  _per_sc_row_blocks(
      s, N, 16,
      lambda off: pltpu.sync_copy(rows0.at[pl.ds(0, 16)],
                                  acc_sp.at[pl.ds(off, 16)]))
  plsc.subcore_barrier()
  base = tile * EPT
  pltpu.sync_copy(src_hbm.at[pl.ds(base, EPT)], srcall)
  pltpu.sync_copy(dst_hbm.at[pl.ds(base, EPT)], dstall)
  for b in range(3):
    pltpu.async_copy(m_hbm.at[srcall.at[pl.ds(b * CHP, CHP)]], rowsb[b],
                     sg[b])

  def chunk(cn, b):
    pltpu.make_async_copy(m_hbm.at[pl.ds(0, CHP)], rowsb[b], sg[b]).wait()
    pltpu.async_copy(rowsb[b], acc_sp.at[dstall.at[pl.ds(cn * CHP, CHP)]],
                     ss[b], add=True)

    @pl.when(cn + 3 < NCHP)
    def _():
      pltpu.make_async_copy(rowsb[b], acc_sp.at[pl.ds(0, CHP)], ss[b]).wait()
      pltpu.async_copy(m_hbm.at[srcall.at[pl.ds((cn + 3) * CHP, CHP)]],
                       rowsb[b], sg[b])

  def group(g, _):
    for b in range(3):
      chunk(g * 3 + b, b)
    return 0

  lax.fori_loop(0, NCHP // 3, group, 0)
  for b in range(NCHP - (NCHP // 3) * 3):
    chunk((NCHP // 3) * 3 + b, b)
  for b in range(3):
    pltpu.make_async_copy(rowsb[b], acc_sp.at[pl.ds(0, CHP)], ss[b]).wait()
  plsc.subcore_barrier()
  _per_sc_row_blocks(
      s, N, 80,
      lambda off: pltpu.sync_copy(acc_sp.at[pl.ds(off, 80)],
                                  out.at[c].at[pl.ds(off, 80)]))


@functools.partial(
    pl.kernel,
    out_type=jax.ShapeDtypeStruct((NC, B, D), jnp.float32),
    mesh=_mesh,
    scratch_types=[
        pltpu.VMEM((80,), jnp.int32),
        pltpu.VMEM((80, D), jnp.float32),
        pltpu.VMEM((B // NS, D), jnp.float32),
        pltpu.VMEM_SHARED((B, D), jnp.float32),
    ],
)
def _sc_atom_agg(h_hbm, gids_hbm, out, sidx, rows, zbuf, acc_sp):
  c, s, tile = _tile_ids()
  bpt = B // NS
  _fill_rows(zbuf, bpt, D, 0.0)
  pltpu.sync_copy(zbuf, acc_sp.at[pl.ds(s * bpt, bpt)])
  plsc.subcore_barrier()

  nblocks = N // 80

  def body(i, _):
    j = i * NW + tile

    @pl.when(j < nblocks)
    def _():
      pltpu.sync_copy(h_hbm.at[pl.ds(j * 80, 80)], rows)
      pltpu.sync_copy(gids_hbm.at[pl.ds(j * 80, 80)], sidx)
      pltpu.sync_copy(rows, acc_sp.at[sidx], add=True)

    return 0

  lax.fori_loop(0, (nblocks + NW - 1) // NW, body, 0)
  plsc.subcore_barrier()
  pltpu.sync_copy(acc_sp.at[pl.ds(s * bpt, bpt)],
                  out.at[c].at[pl.ds(s * bpt, bpt)])


_IPT = (L * B) // NW


@functools.partial(
    pl.kernel,
    out_type=(
        jax.ShapeDtypeStruct((L * B, 128), jnp.int32),
        jax.ShapeDtypeStruct((L * B, MP), jnp.float32),
    ),
    mesh=_mesh,
    scratch_types=[
        pltpu.VMEM((_IPT,), jnp.int32),
        pltpu.VMEM((_IPT,), jnp.int32),
        pltpu.VMEM((_IPT, 128), jnp.int32),
        pltpu.VMEM((_IPT, MP), jnp.float32),
        pltpu.SemaphoreType.DMA,
    ],
)
def _sc_kg_stage1(idflat, idhalf, nm2_hbm, mfea_hbm, out_nei, out_item, idbuf,
                  idhbuf, nmrows, mrows, sem):
  _, _, tile = _tile_ids()
  i0 = tile * _IPT
  pltpu.sync_copy(idflat.at[pl.ds(i0, _IPT)], idbuf)
  pltpu.sync_copy(idhalf.at[pl.ds(i0, _IPT)], idhbuf)
  pltpu.async_copy(nm2_hbm.at[idhbuf], nmrows, sem).wait()
  pltpu.sync_copy(nmrows, out_nei.at[pl.ds(i0, _IPT)])
  pltpu.async_copy(mfea_hbm.at[idbuf], mrows, sem).wait()
  pltpu.sync_copy(mrows, out_item.at[pl.ds(i0, _IPT)])


_NFLAT = L * B * K
_NPT = _NFLAT // NW
_CH2 = 128
_NCH2 = _NPT // _CH2
_ACC2 = L * B


@functools.partial(
    pl.kernel,
    out_type=jax.ShapeDtypeStruct((NC, L * B, MP), jnp.float32),
    mesh=_mesh,
    scratch_types=[
        pltpu.VMEM((_NPT,), jnp.int32),
        pltpu.VMEM((_NPT,), jnp.int32),
        pltpu.VMEM((_CH2, MP), jnp.float32),
        pltpu.VMEM((_CH2, MP), jnp.float32),
        pltpu.VMEM((_CH2, MP), jnp.float32),
        pltpu.VMEM_SHARED((_ACC2, MP), jnp.float32),
        pltpu.SemaphoreType.DMA,
        pltpu.SemaphoreType.DMA,
        pltpu.SemaphoreType.DMA,
        pltpu.SemaphoreType.DMA,
        pltpu.SemaphoreType.DMA,
        pltpu.SemaphoreType.DMA,
    ],
)
def _sc_kg_stage2(neiflat, pat_hbm, mfea_hbm, out, gall, pall, rows0, rows1,
                  rows2, acc_sp, sg0, sg1, sg2, ss0, ss1, ss2):
  c, s, tile = _tile_ids()
  rowsb = (rows0, rows1, rows2)
  sg = (sg0, sg1, sg2)
  ss = (ss0, ss1, ss2)
  _fill_rows(rows0, 8, MP, 0.0)
  _per_sc_row_blocks(
      s, _ACC2, 8,
      lambda off: pltpu.sync_copy(rows0.at[pl.ds(0, 8)],
                                  acc_sp.at[pl.ds(off, 8)]))
  plsc.subcore_barrier()
  base = tile * _NPT
  pltpu.sync_copy(neiflat.at[pl.ds(base, _NPT)], gall)
  pltpu.sync_copy(pat_hbm.at[pl.ds(base, _NPT)], pall)
  for b in range(3):
    pltpu.async_copy(mfea_hbm.at[gall.at[pl.ds(b * _CH2, _CH2)]], rowsb[b],
                     sg[b])

  def chunk(cn, b):
    pltpu.make_async_copy(mfea_hbm.at[pl.ds(0, _CH2)], rowsb[b], sg[b]).wait()
    pltpu.async_copy(rowsb[b], acc_sp.at[pall.at[pl.ds(cn * _CH2, _CH2)]],
                     ss[b], add=True)

    @pl.when(cn + 3 < _NCH2)
    def _():
      pltpu.make_async_copy(rowsb[b], acc_sp.at[pl.ds(0, _CH2)], ss[b]).wait()
      pltpu.async_copy(mfea_hbm.at[gall.at[pl.ds((cn + 3) * _CH2, _CH2)]],
                       rowsb[b], sg[b])

  def group(g, _):
    for b in range(3):
      chunk(g * 3 + b, b)
    return 0

  lax.fori_loop(0, _NCH2 // 3, group, 0)
  for b in range(_NCH2 - (_NCH2 // 3) * 3):
    chunk((_NCH2 // 3) * 3 + b, b)
  for b in range(3):
    pltpu.make_async_copy(rowsb[b], acc_sp.at[pl.ds(0, _CH2)], ss[b]).wait()
  plsc.subcore_barrier()
  rpt = (L * B) // NS
  pltpu.sync_copy(acc_sp.at[pl.ds(s * rpt, rpt)],
                  out.at[c].at[pl.ds(s * rpt, rpt)])


_MB = 2000


def _tc_embed_body(mf_ref, fe_ref, out_ref):
  out_ref[...] = jnp.dot(mf_ref[...], fe_ref[...],
                         preferred_element_type=jnp.float32)


def _tc_embed(all_mfea, fea_emb):
  return pl.pallas_call(
      _tc_embed_body,
      grid=(M // _MB,),
      in_specs=[
          pl.BlockSpec((_MB, 40), lambda i: (i, 0)),
          pl.BlockSpec((40, D), lambda i: (0, 0)),
      ],
      out_specs=pl.BlockSpec((_MB, D), lambda i: (i, 0)),
      out_shape=jax.ShapeDtypeStruct((M, D), jnp.float32),
  )(all_mfea, fea_emb)


def _tc_prep_body(nf_ref, degp_ref, w0_ref, m1_ref, ns_ref, nd_ref):
  dsrc = jnp.sum(degp_ref[:, 0, :], axis=0)
  ddst = jnp.sum(degp_ref[:, 1, :], axis=0)
  ns = jax.lax.rsqrt(jnp.maximum(dsrc, 1.0))[:, None]
  nd = jax.lax.rsqrt(jnp.maximum(ddst, 1.0))[:, None]
  iota = lax.broadcasted_iota(jnp.int32, (N, F), 1)
  h0 = jnp.zeros((N, F), jnp.float32)
  for j in range(4):
    h0 = h0 + (nf_ref[:, j:j + 1] == iota).astype(jnp.float32)
  m1_ref[...] = jnp.dot(h0 * ns, w0_ref[...],
                        preferred_element_type=jnp.float32)
  ns_ref[...] = ns
  nd_ref[...] = nd


def _tc_prep(node_feature, degp, w0):
  return pl.pallas_call(
      _tc_prep_body,
      out_shape=[
          jax.ShapeDtypeStruct((N, D), jnp.float32),
          jax.ShapeDtypeStruct((N, 1), jnp.float32),
          jax.ShapeDtypeStruct((N, 1), jnp.float32),
      ],
  )(node_feature, degp, w0)


def _tc_mid_body(aggp_ref, nd_ref, ns_ref, b0_ref, w1_ref, m2_ref):
  agg = aggp_ref[0] + aggp_ref[1]
  h1 = jnp.maximum(agg * nd_ref[...] + b0_ref[...], 0.0)
  m2_ref[...] = jnp.dot(h1 * ns_ref[...], w1_ref[...],
                        preferred_element_type=jnp.float32)


def _tc_mid(aggp, nd, ns, b0, w1):
  return pl.pallas_call(
      _tc_mid_body,
      out_shape=jax.ShapeDtypeStruct((N, D), jnp.float32),
  )(aggp, nd, ns, b0, w1)


def _tc_h2_body(aggp_ref, nd_ref, b1_ref, h2_ref, fac_ref):
  agg = aggp_ref[0] + aggp_ref[1]
  h2 = agg * nd_ref[...] + b1_ref[...]
  h2_ref[...] = h2
  nsum = jnp.sum(jnp.sqrt(jnp.sum(h2 * h2, axis=1)))
  fac_ref[...] = jnp.full((1, 1), jnp.sqrt(jnp.float32(D)) * N / nsum)


def _tc_h2(aggp, nd, b1):
  return pl.pallas_call(
      _tc_h2_body,
      out_shape=[
          jax.ShapeDtypeStruct((N, D), jnp.float32),
          jax.ShapeDtypeStruct((1, 1), jnp.float32),
      ],
  )(aggp, nd, b1)


def _tc_nm2_body(nm3_ref, out_ref):
  out_ref[...] = jnp.concatenate([nm3_ref[:, 0, :], nm3_ref[:, 1, :]], axis=1)


def _tc_nm2(neighbor_matrix):
  return pl.pallas_call(
      _tc_nm2_body,
      grid=(M // 2000,),
      in_specs=[pl.BlockSpec((1000, 2, K), lambda i: (i, 0, 0))],
      out_specs=pl.BlockSpec((1000, 2 * K), lambda i: (i, 0)),
      out_shape=jax.ShapeDtypeStruct(((M * K) // 128, 128), jnp.int32),
  )(neighbor_matrix.reshape(M // 2, 2, K))


def _tc_compact_body(nei_ref, par_ref, out_ref):
  sel = par_ref[...] == 1
  out_ref[...] = jnp.where(sel, nei_ref[:, K:], nei_ref[:, :K])


def _tc_compact(nei_rows, par):
  return pl.pallas_call(
      _tc_compact_body,
      out_shape=jax.ShapeDtypeStruct((L * B, K), jnp.int32),
  )(nei_rows, par)


_BB = 256


def _tc_final_body(molap_ref, factor_ref, item_ref, neip_ref, cw_ref,
                   cb_ref, emb_ref, a_ref, x_ref):
  mol_r = (item_ref[0] + item_ref[1] + neip_ref[0, 0] + neip_ref[0, 1] +
           neip_ref[1, 0] + neip_ref[1, 1]) * 0.5
  mol_a = (molap_ref[0] + molap_ref[1]) * factor_ref[0, 0]
  emb_ref[...] = mol_a + mol_r
  a_ref[...] = mol_a
  x_ref[...] = jnp.dot(mol_r, cw_ref[...],
                       preferred_element_type=jnp.float32) + cb_ref[...]


def _tc_final(molap, factor, item2, neip, cw, cb):
  return pl.pallas_call(
      _tc_final_body,
      grid=(B // _BB,),
      in_specs=[
          pl.BlockSpec((NC, _BB, D), lambda i: (0, i, 0)),
          pl.BlockSpec((1, 1), lambda i: (0, 0)),
          pl.BlockSpec((L, _BB, MP), lambda i: (0, i, 0)),
          pl.BlockSpec((NC, L, _BB, MP), lambda i: (0, 0, i, 0)),
          pl.BlockSpec((D, D), lambda i: (0, 0)),
          pl.BlockSpec((1, D), lambda i: (0, 0)),
      ],
      out_specs=[
          pl.BlockSpec((_BB, D), lambda i: (i, 0)),
          pl.BlockSpec((_BB, D), lambda i: (i, 0)),
          pl.BlockSpec((_BB, D), lambda i: (i, 0)),
      ],
      out_shape=[
          jax.ShapeDtypeStruct((B, D), jnp.float32),
          jax.ShapeDtypeStruct((B, D), jnp.float32),
          jax.ShapeDtypeStruct((B, D), jnp.float32),
      ],
  )(molap, factor, item2, neip, cw, cb)


@jax.jit
def kernel(node_feature, edge_index, graph_ids, id_list, id_list_target,
           neighbor_matrix, relation_matrix, all_mfea, all_rfea,
           W0, b0, W1, b1, fea_emb, rel_emb, crosscl_w, crosscl_b):
  del id_list_target, relation_matrix, all_rfea, rel_emb

  src = edge_index[0].astype(jnp.int32)
  dst = edge_index[1].astype(jnp.int32)
  gids = graph_ids.astype(jnp.int32)
  idflat = id_list.astype(jnp.int32).reshape(L * B)
  idhalf = idflat // 2
  nm2 = neighbor_matrix.astype(jnp.int32).reshape((M * K) // 128, 128)
  mfea_emb = _tc_embed(all_mfea, fea_emb)
  pat = jnp.repeat(jnp.arange(L * B, dtype=jnp.int32), K)

  degp = _sc_degrees(src, dst)
  m1, ns, nd = _tc_prep(node_feature.astype(jnp.int32), degp, W0)
  agg1p = _sc_edge_pass(m1, src, dst)
  m2 = _tc_mid(agg1p, nd, ns, b0.reshape(1, D), W1)
  agg2p = _sc_edge_pass(m2, src, dst)
  h2, factor = _tc_h2(agg2p, nd, b1.reshape(1, D))
  molap = _sc_atom_agg(h2, gids)

  nei_rows, item_rows = _sc_kg_stage1(idflat, idhalf, nm2, mfea_emb)
  nei_compact = _tc_compact(nei_rows, (idflat % 2).reshape(L * B, 1))
  neip = _sc_kg_stage2(nei_compact.reshape(_NFLAT), pat, mfea_emb)

  mol_emb, mol_a, mol_x = _tc_final(
      molap, factor, item_rows.reshape(L, B, MP),
      neip.reshape(NC, L, B, MP), crosscl_w, crosscl_b.reshape(1, D))
  return (mol_emb, mol_a, mol_x)
